# baseline (speedup 1.0000x reference)
import functools

import jax
import jax.numpy as jnp
from jax import lax
from jax.experimental import pallas as pl
from jax.experimental.pallas import tpu as pltpu
from jax.experimental.pallas import tpu_sc as plsc

B = 8192
PRED_DIM = 1024
PROTO_DIM = 256
NUM_PROTOS = 8192

BR = 1024
BC = 2048
NI = B // BR
NJ = NUM_PROTOS // BC
NSTRIP = BC // 64

_INV_SQRT2 = 0.7071067811865476


def _fused_kernel(temp_ref, z_ref, w1_ref, b1_ref, w2_ref, b2_ref, proto_ref,
                  recent_ref, support_ref,
                  sim_ref, bs_ref, bm_ref, dr_ref, scal_ref,
                  pn_ref, zn_ref, vacc_ref, m_ref, s_ref, am_ref):
    i = pl.program_id(0)
    j = pl.program_id(1)
    inv_t = 1.0 / (jnp.abs(temp_ref[0, 0]) + 0.1)
    t = jnp.sum(recent_ref[...])
    rt = t + 1e-8

    @pl.when(i == 0)
    def _():
        p = proto_ref[...]
        pnorm = jnp.sqrt(jnp.sum(p * p, axis=1, keepdims=True))
        pn = p / jnp.maximum(pnorm, 1e-12)
        pn_ref[pl.ds(j * BC, BC), :] = pn
        rd_blk = recent_ref[0:1, pl.ds(j * BC, BC)] / rt
        vpart = lax.dot_general(pn, rd_blk, (((0,), (1,)), ((), ())),
                                preferred_element_type=jnp.float32)

        @pl.when(j == 0)
        def _():
            vacc_ref[...] = jnp.zeros(vacc_ref.shape, jnp.float32)

        vacc_ref[...] += vpart

    @pl.when(j == 0)
    def _():
        x = z_ref[...]
        h = jnp.dot(x, w1_ref[...], preferred_element_type=jnp.float32) + b1_ref[...]
        h = 0.5 * h * (1.0 + lax.erf(h * _INV_SQRT2))
        zp = jnp.dot(h, w2_ref[...], preferred_element_type=jnp.float32) + b2_ref[...]
        n = jnp.sqrt(jnp.sum(zp * zp, axis=1, keepdims=True))
        zn_ref[...] = zp / jnp.maximum(n, 1e-12)
        m_ref[...] = jnp.full(m_ref.shape, -1e30, jnp.float32)
        s_ref[...] = jnp.zeros(s_ref.shape, jnp.float32)
        am_ref[...] = jnp.zeros(am_ref.shape, jnp.int32)

    zn = zn_ref[...]
    sim = lax.dot_general(zn, pn_ref[pl.ds(j * BC, BC), :],
                          (((1,), (1,)), ((), ())),
                          preferred_element_type=jnp.float32) * inv_t
    sim_ref[...] = sim
    lm = jnp.max(sim, axis=1, keepdims=True)

    e2 = jnp.exp(2.0 * sim)
    ones_row = jnp.ones((1, BC), jnp.float32)
    s_ref[...] += lax.dot_general(e2, ones_row, (((1,), (1,)), ((), ())),
                                  preferred_element_type=jnp.float32)
    onehot = jnp.where(sim == lm, 1.0, 0.0)
    rows16 = lax.broadcasted_iota(jnp.int32, (BC, NSTRIP), 0)
    scol16 = lax.broadcasted_iota(jnp.int32, (BC, NSTRIP), 1)
    pw = lax.bitcast_convert_type((127 - (rows16 & 63)) << 23, jnp.float32)
    w_strip = jnp.where((rows16 >> 6) == scol16, pw, 0.0)
    red = lax.dot_general(onehot, w_strip, (((1,), (0,)), ((), ())),
                          preferred_element_type=jnp.float32)
    strip_idx = lax.broadcasted_iota(jnp.int32, (BR, NSTRIP), 1)
    strip_sel = jnp.min(jnp.where(red > 0.0, strip_idx, jnp.int32(NSTRIP)),
                        axis=1, keepdims=True)
    win = jnp.sum(jnp.where(strip_idx == strip_sel, red, 0.0),
                  axis=1, keepdims=True)
    ebits = lax.shift_right_logical(
        lax.bitcast_convert_type(win, jnp.int32), 23) & 255
    la = strip_sel * 64 + (127 - ebits) + j * BC
    m_old = m_ref[...]
    am_ref[...] = jnp.where(lm > m_old, la, am_ref[...])
    m_ref[...] = jnp.maximum(m_old, lm)

    @pl.when(j == NJ - 1)
    def _():
        bs_ref[...] = m_ref[...]
        bm_ref[...] = am_ref[...]
        dot = lax.dot_general(zn, vacc_ref[...], (((1,), (0,)), ((), ())),
                              preferred_element_type=jnp.float32) * inv_t
        rd = recent_ref[...] / rt
        c1 = jnp.sum(rd * jnp.log(rd))
        lse = jnp.log(s_ref[...])
        dr_ref[...] = c1 - 2.0 * dot + (t / rt) * lse
        scal_ref[0, 0] = c1
        scal_ref[0, 1] = t / rt
        scal_ref[0, 2] = inv_t
        scal_ref[0, 3] = jnp.max(support_ref[...]) + 1e-8


_fused_call = pl.pallas_call(
    _fused_kernel,
    grid=(NI, NJ),
    in_specs=[
        pl.BlockSpec(memory_space=pltpu.SMEM),
        pl.BlockSpec((BR, PRED_DIM), lambda i, j: (i, 0)),
        pl.BlockSpec((PRED_DIM, 2 * PROTO_DIM), lambda i, j: (0, 0)),
        pl.BlockSpec((1, 2 * PROTO_DIM), lambda i, j: (0, 0)),
        pl.BlockSpec((2 * PROTO_DIM, PROTO_DIM), lambda i, j: (0, 0)),
        pl.BlockSpec((1, PROTO_DIM), lambda i, j: (0, 0)),
        pl.BlockSpec((BC, PROTO_DIM), lambda i, j: (jnp.where(i == 0, j, 0), 0)),
        pl.BlockSpec((1, NUM_PROTOS), lambda i, j: (0, 0)),
        pl.BlockSpec((1, NUM_PROTOS), lambda i, j: (0, 0)),
    ],
    out_specs=[
        pl.BlockSpec((BR, BC), lambda i, j: (i, j)),
        pl.BlockSpec((BR, 1), lambda i, j: (i, 0)),
        pl.BlockSpec((BR, 1), lambda i, j: (i, 0)),
        pl.BlockSpec((BR, 1), lambda i, j: (i, 0)),
        pl.BlockSpec(memory_space=pltpu.SMEM),
    ],
    out_shape=[
        jax.ShapeDtypeStruct((B, NUM_PROTOS), jnp.float32),
        jax.ShapeDtypeStruct((B, 1), jnp.float32),
        jax.ShapeDtypeStruct((B, 1), jnp.int32),
        jax.ShapeDtypeStruct((B, 1), jnp.float32),
        jax.ShapeDtypeStruct((1, 4), jnp.float32),
    ],
    scratch_shapes=[
        pltpu.VMEM((NUM_PROTOS, PROTO_DIM), jnp.float32),
        pltpu.VMEM((BR, PROTO_DIM), jnp.float32),
        pltpu.VMEM((PROTO_DIM, 1), jnp.float32),
        pltpu.VMEM((BR, 1), jnp.float32),
        pltpu.VMEM((BR, 1), jnp.float32),
        pltpu.VMEM((BR, 1), jnp.int32),
    ],
)


_NC = 2
_NS = 16
_NL = 16
_NW = _NC * _NS
_RPW = B // _NW


@functools.cache
def _sc_epilogue_call():
    @functools.partial(
        pl.kernel,
        mesh=plsc.VectorSubcoreMesh(core_axis_name="c", subcore_axis_name="s",
                                    num_cores=_NC),
        out_type=(jax.ShapeDtypeStruct((B,), jnp.float32),
                  jax.ShapeDtypeStruct((B,), jnp.float32)),
        scratch_types=[
            pltpu.VMEM((_RPW,), jnp.int32),
            pltpu.VMEM((_RPW,), jnp.float32),
            pltpu.VMEM((_NL,), jnp.float32),
            pltpu.VMEM((_RPW,), jnp.float32),
            pltpu.VMEM((_RPW,), jnp.float32),
            pltpu.VMEM((_RPW,), jnp.float32),
            pltpu.VMEM((_RPW,), jnp.float32),
            pltpu.SemaphoreType.DMA,
            pltpu.SemaphoreType.DMA,
        ],
    )
    def _sc_epilogue(bs_hbm, bm_hbm, sup_hbm, qual_hbm, sv_hbm,
                     sd_hbm, fam_hbm,
                     idx_v, bs_v, sv_v, ms_v, mq_v, sd_v, fam_v,
                     sem0, sem1):
        wid = lax.axis_index("s") * _NC + lax.axis_index("c")
        base = wid * _RPW
        pltpu.sync_copy(bm_hbm.at[pl.ds(base, _RPW)], idx_v)
        pltpu.sync_copy(bs_hbm.at[pl.ds(base, _RPW)], bs_v)
        pltpu.sync_copy(sv_hbm, sv_v)
        cp0 = pltpu.async_copy(sup_hbm.at[idx_v], ms_v, sem0)
        cp1 = pltpu.async_copy(qual_hbm.at[idx_v], mq_v, sem1)
        cp0.wait()
        cp1.wait()
        maxsup = sv_v[...]
        for k in range(_RPW // _NL):
            sl = pl.ds(k * _NL, _NL)
            bs = bs_v[sl]
            ms = ms_v[sl]
            mq = mq_v[sl]
            sig = 1.0 / (1.0 + jnp.exp(-bs))
            sd = jnp.minimum(jnp.maximum(ms / maxsup * mq * sig, 0.1), 1.0)
            fam = (1.0 / (1.0 + jnp.exp(-(bs - 0.3) * 3.0))) * sd
            sd_v[sl] = sd
            fam_v[sl] = fam
        pltpu.sync_copy(sd_v, sd_hbm.at[pl.ds(base, _RPW)])
        pltpu.sync_copy(fam_v, fam_hbm.at[pl.ds(base, _RPW)])

    return _sc_epilogue


def kernel(z_pred, prototypes, W1, b1, W2, b2, support_counts,
           recent_assignments, prototype_quality, temperature):
    temp = jnp.asarray(temperature, jnp.float32).reshape(1, 1)
    sim, bs, bm, dr, scal = _fused_call(
        temp, z_pred, W1, b1.reshape(1, -1), W2, b2.reshape(1, -1),
        prototypes, recent_assignments.reshape(1, -1),
        support_counts.reshape(1, -1))
    bm_flat = bm.reshape(-1)
    sv = jnp.broadcast_to(scal[0, 3], (_NL,))
    sd, fam = _sc_epilogue_call()(bs.reshape(-1), bm_flat, support_counts,
                                  prototype_quality, sv)
    return (sim, sd, fam, dr.reshape(-1), bm_flat)


# speedup vs baseline: 1.1510x; 1.1510x over previous
import functools

import jax
import jax.numpy as jnp
from jax import lax
from jax.experimental import pallas as pl
from jax.experimental.pallas import tpu as pltpu
from jax.experimental.pallas import tpu_sc as plsc

B = 8192
PRED_DIM = 1024
PROTO_DIM = 256
NUM_PROTOS = 8192

BR = 1024
BC = 2048
NI = B // BR
NJ = NUM_PROTOS // BC
NSTRIP = BC // 64

_INV_SQRT2 = 0.7071067811865476


def _mlp_kernel(temp_ref, z_ref, w1_ref, b1_ref, w2_ref, b2_ref, proto_ref,
                recent_ref, support_ref,
                zn_ref, pn_ref, v_ref, scal_ref, vacc_ref):
    i = pl.program_id(0)
    inv_t = 1.0 / (jnp.abs(temp_ref[0, 0]) + 0.1)
    x = z_ref[...]
    h = jnp.dot(x, w1_ref[...], preferred_element_type=jnp.float32) + b1_ref[...]
    h = 0.5 * h * (1.0 + lax.erf(h * _INV_SQRT2))
    zp = jnp.dot(h, w2_ref[...], preferred_element_type=jnp.float32) + b2_ref[...]
    n = jnp.sqrt(jnp.sum(zp * zp, axis=1, keepdims=True))
    zn_ref[...] = zp / jnp.maximum(n, 1e-12)
    p = proto_ref[...]
    pnorm = jnp.sqrt(jnp.sum(p * p, axis=1, keepdims=True))
    pn = p / jnp.maximum(pnorm, 1e-12)
    pn_ref[...] = pn
    rec = recent_ref[...]
    t = jnp.sum(rec)
    rt = t + 1e-8
    rd = rec / rt
    rd_blk = recent_ref[0:1, pl.ds(i * BR, BR)] / rt
    vpart = lax.dot_general(pn, rd_blk, (((0,), (1,)), ((), ())),
                            preferred_element_type=jnp.float32)

    @pl.when(i == 0)
    def _():
        vacc_ref[...] = jnp.zeros(vacc_ref.shape, jnp.float32)

    vacc_ref[...] += vpart

    @pl.when(i == NI - 1)
    def _():
        v_ref[...] = vacc_ref[...]

    scal_ref[0, 0] = jnp.sum(rd * jnp.log(rd))
    scal_ref[0, 1] = t / rt
    scal_ref[0, 2] = inv_t
    scal_ref[0, 3] = jnp.max(support_ref[...]) + 1e-8


_mlp_call = pl.pallas_call(
    _mlp_kernel,
    grid=(NI,),
    in_specs=[
        pl.BlockSpec(memory_space=pltpu.SMEM),
        pl.BlockSpec((BR, PRED_DIM), lambda i: (i, 0)),
        pl.BlockSpec((PRED_DIM, 2 * PROTO_DIM), lambda i: (0, 0)),
        pl.BlockSpec((1, 2 * PROTO_DIM), lambda i: (0, 0)),
        pl.BlockSpec((2 * PROTO_DIM, PROTO_DIM), lambda i: (0, 0)),
        pl.BlockSpec((1, PROTO_DIM), lambda i: (0, 0)),
        pl.BlockSpec((BR, PROTO_DIM), lambda i: (i, 0)),
        pl.BlockSpec((1, NUM_PROTOS), lambda i: (0, 0)),
        pl.BlockSpec((1, NUM_PROTOS), lambda i: (0, 0)),
    ],
    out_specs=[
        pl.BlockSpec((BR, PROTO_DIM), lambda i: (i, 0)),
        pl.BlockSpec((BR, PROTO_DIM), lambda i: (i, 0)),
        pl.BlockSpec((PROTO_DIM, 1), lambda i: (0, 0)),
        pl.BlockSpec(memory_space=pltpu.SMEM),
    ],
    out_shape=[
        jax.ShapeDtypeStruct((B, PROTO_DIM), jnp.float32),
        jax.ShapeDtypeStruct((NUM_PROTOS, PROTO_DIM), jnp.float32),
        jax.ShapeDtypeStruct((PROTO_DIM, 1), jnp.float32),
        jax.ShapeDtypeStruct((1, 4), jnp.float32),
    ],
    scratch_shapes=[pltpu.VMEM((PROTO_DIM, 1), jnp.float32)],
)


def _sim_kernel(scal_ref, z_ref, p_ref, v_ref,
                sim_ref, bs_ref, bm_ref, dr_ref,
                m_ref, s_ref, am_ref, dot_ref):
    j = pl.program_id(1)
    inv_t = scal_ref[0, 2]
    sim = lax.dot_general(z_ref[...], p_ref[...], (((1,), (1,)), ((), ())),
                          preferred_element_type=jnp.float32) * inv_t
    sim_ref[...] = sim
    lm = jnp.max(sim, axis=1, keepdims=True)

    @pl.when(j == 0)
    def _():
        m_ref[...] = jnp.full(m_ref.shape, -1e30, jnp.float32)
        s_ref[...] = jnp.zeros(s_ref.shape, jnp.float32)
        am_ref[...] = jnp.zeros(am_ref.shape, jnp.int32)
        dot_ref[...] = lax.dot_general(z_ref[...], v_ref[...],
                                       (((1,), (0,)), ((), ())),
                                       preferred_element_type=jnp.float32) * inv_t

    e2 = jnp.exp(2.0 * sim)
    ones_row = jnp.ones((1, BC), jnp.float32)
    s_ref[...] += lax.dot_general(e2, ones_row, (((1,), (1,)), ((), ())),
                                  preferred_element_type=jnp.float32)
    onehot = jnp.where(sim == lm, 1.0, 0.0)
    rows16 = lax.broadcasted_iota(jnp.int32, (BC, NSTRIP), 0)
    scol16 = lax.broadcasted_iota(jnp.int32, (BC, NSTRIP), 1)
    pw = lax.bitcast_convert_type((127 - (rows16 & 63)) << 23, jnp.float32)
    w_strip = jnp.where((rows16 >> 6) == scol16, pw, 0.0)
    red = lax.dot_general(onehot, w_strip, (((1,), (0,)), ((), ())),
                          preferred_element_type=jnp.float32)
    strip_idx = lax.broadcasted_iota(jnp.int32, (BR, NSTRIP), 1)
    strip_sel = jnp.min(jnp.where(red > 0.0, strip_idx, jnp.int32(NSTRIP)),
                        axis=1, keepdims=True)
    win = jnp.sum(jnp.where(strip_idx == strip_sel, red, 0.0),
                  axis=1, keepdims=True)
    ebits = lax.shift_right_logical(
        lax.bitcast_convert_type(win, jnp.int32), 23) & 255
    la = strip_sel * 64 + (127 - ebits) + j * BC
    m_old = m_ref[...]
    am_ref[...] = jnp.where(lm > m_old, la, am_ref[...])
    m_ref[...] = jnp.maximum(m_old, lm)

    @pl.when(j == NJ - 1)
    def _():
        bs_ref[...] = m_ref[...]
        bm_ref[...] = am_ref[...]
        lse = jnp.log(s_ref[...])
        dr_ref[...] = scal_ref[0, 0] - 2.0 * dot_ref[...] + scal_ref[0, 1] * lse


_sim_call = pl.pallas_call(
    _sim_kernel,
    grid=(NI, NJ),
    in_specs=[
        pl.BlockSpec(memory_space=pltpu.SMEM),
        pl.BlockSpec((BR, PROTO_DIM), lambda i, j: (i, 0)),
        pl.BlockSpec((BC, PROTO_DIM), lambda i, j: (j, 0)),
        pl.BlockSpec((PROTO_DIM, 1), lambda i, j: (0, 0)),
    ],
    out_specs=[
        pl.BlockSpec((BR, BC), lambda i, j: (i, j)),
        pl.BlockSpec((BR, 1), lambda i, j: (i, 0)),
        pl.BlockSpec((BR, 1), lambda i, j: (i, 0)),
        pl.BlockSpec((BR, 1), lambda i, j: (i, 0)),
    ],
    out_shape=[
        jax.ShapeDtypeStruct((B, NUM_PROTOS), jnp.float32),
        jax.ShapeDtypeStruct((B, 1), jnp.float32),
        jax.ShapeDtypeStruct((B, 1), jnp.int32),
        jax.ShapeDtypeStruct((B, 1), jnp.float32),
    ],
    scratch_shapes=[
        pltpu.VMEM((BR, 1), jnp.float32),
        pltpu.VMEM((BR, 1), jnp.float32),
        pltpu.VMEM((BR, 1), jnp.int32),
        pltpu.VMEM((BR, 1), jnp.float32),
    ],
)


_NC = 2
_NS = 16
_NL = 16
_NW = _NC * _NS
_RPW = B // _NW


@functools.cache
def _sc_epilogue_call():
    @functools.partial(
        pl.kernel,
        mesh=plsc.VectorSubcoreMesh(core_axis_name="c", subcore_axis_name="s",
                                    num_cores=_NC),
        out_type=(jax.ShapeDtypeStruct((B,), jnp.float32),
                  jax.ShapeDtypeStruct((B,), jnp.float32)),
        scratch_types=[
            pltpu.VMEM((_RPW,), jnp.int32),
            pltpu.VMEM((_RPW,), jnp.float32),
            pltpu.VMEM((_NL,), jnp.float32),
            pltpu.VMEM((_RPW,), jnp.float32),
            pltpu.VMEM((_RPW,), jnp.float32),
            pltpu.VMEM((_RPW,), jnp.float32),
            pltpu.VMEM((_RPW,), jnp.float32),
            pltpu.SemaphoreType.DMA,
            pltpu.SemaphoreType.DMA,
        ],
    )
    def _sc_epilogue(bs_hbm, bm_hbm, sup_hbm, qual_hbm, sv_hbm,
                     sd_hbm, fam_hbm,
                     idx_v, bs_v, sv_v, ms_v, mq_v, sd_v, fam_v,
                     sem0, sem1):
        wid = lax.axis_index("s") * _NC + lax.axis_index("c")
        base = wid * _RPW
        pltpu.sync_copy(bm_hbm.at[pl.ds(base, _RPW)], idx_v)
        pltpu.sync_copy(bs_hbm.at[pl.ds(base, _RPW)], bs_v)
        pltpu.sync_copy(sv_hbm, sv_v)
        cp0 = pltpu.async_copy(sup_hbm.at[idx_v], ms_v, sem0)
        cp1 = pltpu.async_copy(qual_hbm.at[idx_v], mq_v, sem1)
        cp0.wait()
        cp1.wait()
        maxsup = sv_v[...]
        for k in range(_RPW // _NL):
            sl = pl.ds(k * _NL, _NL)
            bs = bs_v[sl]
            ms = ms_v[sl]
            mq = mq_v[sl]
            sig = 1.0 / (1.0 + jnp.exp(-bs))
            sd = jnp.minimum(jnp.maximum(ms / maxsup * mq * sig, 0.1), 1.0)
            fam = (1.0 / (1.0 + jnp.exp(-(bs - 0.3) * 3.0))) * sd
            sd_v[sl] = sd
            fam_v[sl] = fam
        pltpu.sync_copy(sd_v, sd_hbm.at[pl.ds(base, _RPW)])
        pltpu.sync_copy(fam_v, fam_hbm.at[pl.ds(base, _RPW)])

    return _sc_epilogue


def kernel(z_pred, prototypes, W1, b1, W2, b2, support_counts,
           recent_assignments, prototype_quality, temperature):
    temp = jnp.asarray(temperature, jnp.float32).reshape(1, 1)
    zn, pn, v, scal = _mlp_call(
        temp, z_pred, W1, b1.reshape(1, -1), W2, b2.reshape(1, -1),
        prototypes, recent_assignments.reshape(1, -1),
        support_counts.reshape(1, -1))
    sim, bs, bm, dr = _sim_call(scal, zn, pn, v)
    bm_flat = bm.reshape(-1)
    sv = jnp.broadcast_to(scal[0, 3], (_NL,))
    sd, fam = _sc_epilogue_call()(bs.reshape(-1), bm_flat, support_counts,
                                  prototype_quality, sv)
    return (sim, sd, fam, dr.reshape(-1), bm_flat)
